# baseline (speedup 1.0000x reference)
import jax
import jax.numpy as jnp
from jax.experimental import pallas as pl
from jax.experimental.pallas import tpu as pltpu

H, D, HID = 16, 64, 1024
EPS = 1e-6
T = 512


def _fused_kernel(hs_ref, Wq_ref, bq_ref, Wkv_ref, bkv_ref,
                  Wo_ref, MZ_ref,
                  out_ref, ktv_ref, zp_ref):
    c = pl.program_id(1)
    f32 = jnp.float32
    bf = jnp.bfloat16
    hs = hs_ref[0]
    q32 = jnp.dot(hs, Wq_ref[...], preferred_element_type=f32) + bq_ref[...]
    qb = q32.astype(bf)
    nd = jnp.dot(qb, MZ_ref[...], preferred_element_type=f32)
    num = nd[:, :HID]
    den = nd[:, HID:]
    attn = num / (jnp.abs(den) + EPS)
    attn_b = attn.astype(bf)
    out_ref[0] = jnp.dot(attn_b, Wo_ref[...], preferred_element_type=f32)
    hs_b = hs.astype(bf)
    kv32 = jnp.dot(hs_b, Wkv_ref[...], preferred_element_type=f32) + bkv_ref[...]
    k32 = kv32[:, :HID]
    zp = jnp.sum(k32, axis=0, keepdims=True)
    kb = k32.astype(bf)
    vb = kv32[:, HID:].astype(bf)
    ktv_full = jax.lax.dot_general(kb, vb, (((0,), (0,)), ((), ())),
                                   preferred_element_type=f32)
    ktv = jnp.concatenate(
        [ktv_full[h * D:(h + 1) * D, h * D:(h + 1) * D] for h in range(H)],
        axis=1)

    @pl.when(c == 0)
    def _init():
        ktv_ref[0] = ktv
        zp_ref[0] = zp

    @pl.when(c != 0)
    def _accum():
        ktv_ref[0] += ktv
        zp_ref[0] += zp


def kernel(hidden_states, M, z, Wq, bq, Wk, bk, Wv, bv, Wo):
    B, S, _ = hidden_states.shape
    NC = S // T
    f32 = jnp.float32
    eye = jnp.eye(H, dtype=f32)
    Mbd = (eye[:, None, :, None] * M[:, :, None, :]).reshape(HID, HID)
    Zseg = jnp.broadcast_to(eye[:, None, :, None] * z[:, :, None, None],
                            (H, D, H, D)).reshape(HID, HID)
    MZ = jnp.concatenate([Mbd, Zseg], axis=1).astype(jnp.bfloat16)
    Wo_b = Wo.astype(jnp.bfloat16)
    bq2 = bq.reshape(1, HID)
    Wkv = jnp.concatenate([Wk, Wv], axis=1).astype(jnp.bfloat16)
    bkv = jnp.concatenate([bk, bv]).reshape(1, 2 * HID)

    wspec = pl.BlockSpec((HID, HID), lambda b, c: (0, 0))
    w2spec = pl.BlockSpec((HID, 2 * HID), lambda b, c: (0, 0))
    bspec = pl.BlockSpec((1, HID), lambda b, c: (0, 0))
    b2spec = pl.BlockSpec((1, 2 * HID), lambda b, c: (0, 0))
    out, ktv, zp = pl.pallas_call(
        _fused_kernel,
        grid=(B, NC),
        in_specs=[
            pl.BlockSpec((1, T, HID), lambda b, c: (b, c, 0)),
            wspec, bspec, w2spec, b2spec,
            wspec, w2spec,
        ],
        out_specs=[
            pl.BlockSpec((1, T, HID), lambda b, c: (b, c, 0)),
            pl.BlockSpec((1, D, HID), lambda b, c: (b, 0, 0)),
            pl.BlockSpec((1, 1, HID), lambda b, c: (b, 0, 0)),
        ],
        out_shape=[
            jax.ShapeDtypeStruct((B, S, HID), f32),
            jax.ShapeDtypeStruct((B, D, HID), f32),
            jax.ShapeDtypeStruct((B, 1, HID), f32),
        ],
        compiler_params=pltpu.CompilerParams(
            dimension_semantics=("parallel", "arbitrary"),
        ),
        name="linear_memory_attention",
    )(hidden_states, Wq, bq2, Wkv, bkv, Wo_b, MZ)

    ktv_sum = ktv[0] + ktv[1]
    M_new = M + jnp.moveaxis(ktv_sum.reshape(D, H, D), 1, 0)
    z_new = z + (zp[0, 0] + zp[1, 0]).reshape(H, D)
    return out, M_new, z_new


# speedup vs baseline: 1.0133x; 1.0133x over previous
import jax
import jax.numpy as jnp
from jax.experimental import pallas as pl
from jax.experimental.pallas import tpu as pltpu

H, D, HID = 16, 64, 1024
EPS = 1e-6
T = 1024


def _fused_kernel(hs_ref, Wq_ref, bq_ref, Wkv_ref, bkv_ref,
                  Wo_ref, MZ_ref,
                  out_ref, ktv_ref, zp_ref):
    c = pl.program_id(1)
    f32 = jnp.float32
    bf = jnp.bfloat16
    hs = hs_ref[0]
    q32 = jnp.dot(hs, Wq_ref[...], preferred_element_type=f32) + bq_ref[...]
    qb = q32.astype(bf)
    nd = jnp.dot(qb, MZ_ref[...], preferred_element_type=f32)
    num = nd[:, :HID]
    den = nd[:, HID:]
    attn = num / (jnp.abs(den) + EPS)
    attn_b = attn.astype(bf)
    out_ref[0] = jnp.dot(attn_b, Wo_ref[...], preferred_element_type=f32)
    hs_b = hs.astype(bf)
    kv32 = jnp.dot(hs_b, Wkv_ref[...], preferred_element_type=f32) + bkv_ref[...]
    k32 = kv32[:, :HID]
    zp = jnp.sum(k32, axis=0, keepdims=True)
    kb = k32.astype(bf)
    vb = kv32[:, HID:].astype(bf)
    ktv_full = jax.lax.dot_general(kb, vb, (((0,), (0,)), ((), ())),
                                   preferred_element_type=f32)
    ktv = jnp.concatenate(
        [ktv_full[h * D:(h + 1) * D, h * D:(h + 1) * D] for h in range(H)],
        axis=1)

    @pl.when(c == 0)
    def _init():
        ktv_ref[0] = ktv
        zp_ref[0] = zp

    @pl.when(c != 0)
    def _accum():
        ktv_ref[0] += ktv
        zp_ref[0] += zp


def kernel(hidden_states, M, z, Wq, bq, Wk, bk, Wv, bv, Wo):
    B, S, _ = hidden_states.shape
    NC = S // T
    f32 = jnp.float32
    eye = jnp.eye(H, dtype=f32)
    Mbd = (eye[:, None, :, None] * M[:, :, None, :]).reshape(HID, HID)
    Zseg = jnp.broadcast_to(eye[:, None, :, None] * z[:, :, None, None],
                            (H, D, H, D)).reshape(HID, HID)
    MZ = jnp.concatenate([Mbd, Zseg], axis=1).astype(jnp.bfloat16)
    Wo_b = Wo.astype(jnp.bfloat16)
    bq2 = bq.reshape(1, HID)
    Wkv = jnp.concatenate([Wk, Wv], axis=1).astype(jnp.bfloat16)
    bkv = jnp.concatenate([bk, bv]).reshape(1, 2 * HID)

    wspec = pl.BlockSpec((HID, HID), lambda b, c: (0, 0))
    w2spec = pl.BlockSpec((HID, 2 * HID), lambda b, c: (0, 0))
    bspec = pl.BlockSpec((1, HID), lambda b, c: (0, 0))
    b2spec = pl.BlockSpec((1, 2 * HID), lambda b, c: (0, 0))
    out, ktv, zp = pl.pallas_call(
        _fused_kernel,
        grid=(B, NC),
        in_specs=[
            pl.BlockSpec((1, T, HID), lambda b, c: (b, c, 0)),
            wspec, bspec, w2spec, b2spec,
            wspec, w2spec,
        ],
        out_specs=[
            pl.BlockSpec((1, T, HID), lambda b, c: (b, c, 0)),
            pl.BlockSpec((1, D, HID), lambda b, c: (b, 0, 0)),
            pl.BlockSpec((1, 1, HID), lambda b, c: (b, 0, 0)),
        ],
        out_shape=[
            jax.ShapeDtypeStruct((B, S, HID), f32),
            jax.ShapeDtypeStruct((B, D, HID), f32),
            jax.ShapeDtypeStruct((B, 1, HID), f32),
        ],
        compiler_params=pltpu.CompilerParams(
            dimension_semantics=("parallel", "arbitrary"),
            vmem_limit_bytes=60 * 1024 * 1024,
        ),
        name="linear_memory_attention",
    )(hidden_states, Wq, bq2, Wkv, bkv, Wo_b, MZ)

    ktv_sum = ktv[0] + ktv[1]
    M_new = M + jnp.moveaxis(ktv_sum.reshape(D, H, D), 1, 0)
    z_new = z + (zp[0, 0] + zp[1, 0]).reshape(H, D)
    return out, M_new, z_new


# speedup vs baseline: 1.0569x; 1.0430x over previous
import jax
import jax.numpy as jnp
from jax.experimental import pallas as pl
from jax.experimental.pallas import tpu as pltpu

H, D, HID = 16, 64, 1024
EPS = 1e-6
T = 512


def _fused_kernel(hs_ref, Wq_ref, bq_ref, Wk_ref, bk_ref, Wv_ref, bv_ref,
                  Wo_ref, MZ_ref,
                  out_ref, ktv_ref, zp_ref,
                  wkb_ref, wvb_ref, wob_ref):
    first = (pl.program_id(0) == 0) & (pl.program_id(1) == 0)
    f32 = jnp.float32
    bf = jnp.bfloat16

    @pl.when(first)
    def _build_weights():
        wkb_ref[...] = Wk_ref[...].astype(bf)
        wvb_ref[...] = Wv_ref[...].astype(bf)
        wob_ref[...] = Wo_ref[...].astype(bf)

    hs = hs_ref[0]
    q32 = jnp.dot(hs, Wq_ref[...], preferred_element_type=f32) + bq_ref[...]
    qb = q32.astype(bf)
    nd = jnp.dot(qb, MZ_ref[...], preferred_element_type=f32)
    num = nd[:, :HID]
    den = nd[:, HID:]
    attn = num / (jnp.abs(den) + EPS)
    attn_b = attn.astype(bf)
    out_ref[0] = jnp.dot(attn_b, wob_ref[...], preferred_element_type=f32)
    hs_b = hs.astype(bf)
    k32 = jnp.dot(hs_b, wkb_ref[...], preferred_element_type=f32) + bk_ref[...]
    zp = jnp.sum(k32, axis=0, keepdims=True)
    kb = k32.astype(bf)
    v32 = jnp.dot(hs_b, wvb_ref[...], preferred_element_type=f32) + bv_ref[...]
    vb = v32.astype(bf)
    ktv_full = jax.lax.dot_general(kb, vb, (((0,), (0,)), ((), ())),
                                   preferred_element_type=f32)
    ktv = jnp.concatenate(
        [ktv_full[h * D:(h + 1) * D, h * D:(h + 1) * D] for h in range(H)],
        axis=0)

    @pl.when(first)
    def _init():
        ktv_ref[...] = ktv
        zp_ref[...] = zp

    @pl.when(jnp.logical_not(first))
    def _accum():
        ktv_ref[...] += ktv
        zp_ref[...] += zp


def kernel(hidden_states, M, z, Wq, bq, Wk, bk, Wv, bv, Wo):
    B, S, _ = hidden_states.shape
    NC = S // T
    f32 = jnp.float32
    bf = jnp.bfloat16
    i = jnp.arange(HID)
    bmask = (i[:, None] // D) == (i[None, :] // D)
    Mt = jnp.tile(M.reshape(HID, D), (1, H))
    Zt = jnp.broadcast_to(z.reshape(HID, 1), (HID, HID))
    MZ = jnp.concatenate([jnp.where(bmask, Mt, 0.0),
                          jnp.where(bmask, Zt, 0.0)], axis=1).astype(bf)
    bq2 = bq.reshape(1, HID)
    bk2 = bk.reshape(1, HID)
    bv2 = bv.reshape(1, HID)

    wspec = pl.BlockSpec((HID, HID), lambda b, c: (0, 0))
    bspec = pl.BlockSpec((1, HID), lambda b, c: (0, 0))
    out, ktv, zp = pl.pallas_call(
        _fused_kernel,
        grid=(B, NC),
        in_specs=[
            pl.BlockSpec((1, T, HID), lambda b, c: (b, c, 0)),
            wspec, bspec, wspec, bspec, wspec, bspec,
            wspec,
            pl.BlockSpec((HID, 2 * HID), lambda b, c: (0, 0)),
        ],
        out_specs=[
            pl.BlockSpec((1, T, HID), lambda b, c: (b, c, 0)),
            pl.BlockSpec((HID, D), lambda b, c: (0, 0)),
            pl.BlockSpec((1, HID), lambda b, c: (0, 0)),
        ],
        out_shape=[
            jax.ShapeDtypeStruct((B, S, HID), f32),
            jax.ShapeDtypeStruct((HID, D), f32),
            jax.ShapeDtypeStruct((1, HID), f32),
        ],
        scratch_shapes=[
            pltpu.VMEM((HID, HID), bf),
            pltpu.VMEM((HID, HID), bf),
            pltpu.VMEM((HID, HID), bf),
        ],
        compiler_params=pltpu.CompilerParams(
            dimension_semantics=("arbitrary", "arbitrary"),
            vmem_limit_bytes=60 * 1024 * 1024,
        ),
        name="linear_memory_attention",
    )(hidden_states, Wq, bq2, Wk, bk2, Wv, bv2, Wo, MZ)

    M_new = M + ktv.reshape(H, D, D)
    z_new = z + zp.reshape(H, D)
    return out, M_new, z_new


# speedup vs baseline: 1.0658x; 1.0084x over previous
import jax
import jax.numpy as jnp
from jax.experimental import pallas as pl
from jax.experimental.pallas import tpu as pltpu

H, D, HID = 16, 64, 1024
EPS = 1e-6
T = 512


def _fused_kernel(hs_ref, Wq_ref, bq_ref, Wk_ref, bk_ref, Wv_ref, bv_ref,
                  Wo_ref, MZ_ref, M2_ref, z_ref,
                  out_ref, mnew_ref, znew_ref,
                  wkb_ref, wvb_ref, wob_ref, ktv_ref, zp_ref):
    first = (pl.program_id(0) == 0) & (pl.program_id(1) == 0)
    last = ((pl.program_id(0) == pl.num_programs(0) - 1)
            & (pl.program_id(1) == pl.num_programs(1) - 1))
    f32 = jnp.float32
    bf = jnp.bfloat16

    @pl.when(first)
    def _build_weights():
        wkb_ref[...] = Wk_ref[...].astype(bf)
        wvb_ref[...] = Wv_ref[...].astype(bf)
        wob_ref[...] = Wo_ref[...].astype(bf)

    hs = hs_ref[0]
    q32 = jnp.dot(hs, Wq_ref[...], preferred_element_type=f32) + bq_ref[...]
    qb = q32.astype(bf)
    nd = jnp.dot(qb, MZ_ref[...], preferred_element_type=f32)
    num = nd[:, :HID]
    den = nd[:, HID:]
    attn = num / (jnp.abs(den) + EPS)
    attn_b = attn.astype(bf)
    out_ref[0] = jnp.dot(attn_b, wob_ref[...], preferred_element_type=f32)
    hs_b = hs.astype(bf)
    k32 = jnp.dot(hs_b, wkb_ref[...], preferred_element_type=f32) + bk_ref[...]
    zp = jnp.sum(k32, axis=0, keepdims=True)
    kb = k32.astype(bf)
    v32 = jnp.dot(hs_b, wvb_ref[...], preferred_element_type=f32) + bv_ref[...]
    vb = v32.astype(bf)
    ktv_full = jax.lax.dot_general(kb, vb, (((0,), (0,)), ((), ())),
                                   preferred_element_type=f32)
    ktv = jnp.concatenate(
        [ktv_full[h * D:(h + 1) * D, h * D:(h + 1) * D] for h in range(H)],
        axis=0)

    @pl.when(first)
    def _init():
        ktv_ref[...] = ktv
        zp_ref[...] = zp

    @pl.when(jnp.logical_not(first))
    def _accum():
        ktv_ref[...] += ktv
        zp_ref[...] += zp

    @pl.when(last)
    def _finalize():
        mnew_ref[...] = M2_ref[...] + ktv_ref[...]
        zp16 = jnp.concatenate(
            [zp_ref[:, h * D:(h + 1) * D] for h in range(H)], axis=0)
        znew_ref[...] = z_ref[...] + zp16


def kernel(hidden_states, M, z, Wq, bq, Wk, bk, Wv, bv, Wo):
    B, S, _ = hidden_states.shape
    NC = S // T
    f32 = jnp.float32
    bf = jnp.bfloat16
    i = jnp.arange(HID)
    bmask = (i[:, None] // D) == (i[None, :] // D)
    MTb = jnp.tile(M.reshape(HID, D).astype(bf), (1, H))
    ZTb = jnp.broadcast_to(z.astype(bf).reshape(HID, 1), (HID, HID))
    zero = jnp.bfloat16(0)
    MZ = jnp.concatenate([jnp.where(bmask, MTb, zero),
                          jnp.where(bmask, ZTb, zero)], axis=1)
    bq2 = bq.reshape(1, HID)
    bk2 = bk.reshape(1, HID)
    bv2 = bv.reshape(1, HID)
    M2 = M.reshape(HID, D)

    wspec = pl.BlockSpec((HID, HID), lambda b, c: (0, 0))
    bspec = pl.BlockSpec((1, HID), lambda b, c: (0, 0))
    out, mnew, znew = pl.pallas_call(
        _fused_kernel,
        grid=(B, NC),
        in_specs=[
            pl.BlockSpec((1, T, HID), lambda b, c: (b, c, 0)),
            wspec, bspec, wspec, bspec, wspec, bspec,
            wspec,
            pl.BlockSpec((HID, 2 * HID), lambda b, c: (0, 0)),
            pl.BlockSpec((HID, D), lambda b, c: (0, 0)),
            pl.BlockSpec((H, D), lambda b, c: (0, 0)),
        ],
        out_specs=[
            pl.BlockSpec((1, T, HID), lambda b, c: (b, c, 0)),
            pl.BlockSpec((HID, D), lambda b, c: (0, 0)),
            pl.BlockSpec((H, D), lambda b, c: (0, 0)),
        ],
        out_shape=[
            jax.ShapeDtypeStruct((B, S, HID), f32),
            jax.ShapeDtypeStruct((HID, D), f32),
            jax.ShapeDtypeStruct((H, D), f32),
        ],
        scratch_shapes=[
            pltpu.VMEM((HID, HID), bf),
            pltpu.VMEM((HID, HID), bf),
            pltpu.VMEM((HID, HID), bf),
            pltpu.VMEM((HID, D), f32),
            pltpu.VMEM((1, HID), f32),
        ],
        compiler_params=pltpu.CompilerParams(
            dimension_semantics=("arbitrary", "arbitrary"),
            vmem_limit_bytes=60 * 1024 * 1024,
        ),
        name="linear_memory_attention",
    )(hidden_states, Wq, bq2, Wk, bk2, Wv, bv2, Wo, MZ, M2, z)

    return out, mnew.reshape(H, D, D), znew
